# baseline (speedup 1.0000x reference)
import functools

import jax
import jax.numpy as jnp
from jax import lax
from jax.experimental import pallas as pl
from jax.experimental.pallas import tpu as pltpu
from jax.experimental.pallas import tpu_sc as plsc

_N = 10000
_E = 320000
_C = 128
_NC = 2
_NS = 16
_NPAD = 10240
_ROWS_PER_TILE = _NPAD // _NS
_EDGES_PER_CORE = _E // _NC
_EDGES_PER_TILE = _EDGES_PER_CORE // _NS
_B = 128
_SEGC = 8
_NSEG = 10
_EPT = _NSEG * _SEGC * _B
_EPAD = _NC * _NS * _EPT


def _segsum_sc(x, src5, dst5):
  mesh = plsc.VectorSubcoreMesh(
      core_axis_name="c", subcore_axis_name="s", num_cores=_NC,
      num_subcores=_NS)

  @functools.partial(
      pl.kernel,
      out_type=jax.ShapeDtypeStruct((_NC, _NPAD, _C), jnp.float32),
      mesh=mesh,
      scratch_types=[
          pltpu.VMEM_SHARED((_NPAD, _C), jnp.float32),
          pltpu.VMEM((_SEGC, _B), jnp.int32),
          pltpu.VMEM((_SEGC, _B), jnp.int32),
          pltpu.VMEM((_SEGC, _B), jnp.int32),
          pltpu.VMEM((_SEGC, _B), jnp.int32),
          pltpu.VMEM((_B, _C), jnp.float32),
          pltpu.VMEM((_B, _C), jnp.float32),
          [pltpu.SemaphoreType.DMA] * 8,
      ],
  )
  def k(x_hbm, src_hbm, dst_hbm, out_hbm, acc,
        sbuf0, sbuf1, dbuf0, dbuf1, rows0, rows1, sems):
    c = lax.axis_index("c")
    s = lax.axis_index("s")
    w = c * _NS + s
    row0 = s * _ROWS_PER_TILE
    sbuf = (sbuf0, sbuf1)
    dbuf = (dbuf0, dbuf1)
    rows = (rows0, rows1)
    issem = (sems[0], sems[1])
    idsem = (sems[2], sems[3])
    gsem = (sems[4], sems[5])
    ssem = (sems[6], sems[7])

    def seg_start(sb, g):
      pltpu.async_copy(src_hbm.at[w, g], sbuf[sb], issem[sb])
      pltpu.async_copy(dst_hbm.at[w, g], dbuf[sb], idsem[sb])

    def seg_wait(sb):
      pltpu.make_async_copy(src_hbm.at[w, 0], sbuf[sb], issem[sb]).wait()
      pltpu.make_async_copy(dst_hbm.at[w, 0], dbuf[sb], idsem[sb]).wait()

    def g_start(b, sb, k_):
      pltpu.async_copy(x_hbm.at[sbuf[sb].at[k_]], rows[b], gsem[b])

    def g_wait(b):
      pltpu.make_async_copy(x_hbm.at[sbuf[0].at[0]], rows[b], gsem[b]).wait()

    def s_start(b, sb, k_):
      pltpu.async_copy(rows[b], acc.at[dbuf[sb].at[k_]], ssem[b], add=True)

    def s_wait(b):
      pltpu.make_async_copy(rows[b], acc.at[dbuf[0].at[0]], ssem[b]).wait()

    seg_start(0, 0)
    seg_start(1, 1)

    zv = jnp.zeros((16,), jnp.float32)

    def zbody(i, carry):
      rows0[i // 8, pl.ds((i % 8) * 16, 16)] = zv
      return carry

    lax.fori_loop(0, _B * _C // 16, zbody, 0)
    for r in range(_ROWS_PER_TILE // _B):
      pltpu.sync_copy(rows0, acc.at[pl.ds(row0 + r * _B, _B)])
    plsc.subcore_barrier()
    seg_wait(0)
    g_start(0, 0, 0)
    g_start(1, 0, 1)

    def body(gg, carry):
      for sb in (0, 1):
        seg = 2 * gg + sb
        for k_ in range(_SEGC):
          b = k_ % 2
          g_wait(b)
          s_start(b, sb, k_)
          s_wait(b)
          if k_ < _SEGC - 2:
            g_start(b, sb, k_ + 2)
          elif k_ == _SEGC - 2:
            @pl.when(seg < _NSEG - 1)
            def _():
              seg_wait(1 - sb)
              g_start(b, 1 - sb, 0)
          else:
            @pl.when(seg < _NSEG - 1)
            def _():
              g_start(b, 1 - sb, 1)
            @pl.when(seg < _NSEG - 2)
            def _():
              seg_start(sb, seg + 2)
      return carry

    lax.fori_loop(0, _NSEG // 2, body, 0)

    plsc.subcore_barrier()
    pltpu.sync_copy(acc.at[pl.ds(row0, _ROWS_PER_TILE)],
                    out_hbm.at[c, pl.ds(row0, _ROWS_PER_TILE)])

  return k(x, src5, dst5)


_RB = 10000
_GRID = _N // _RB


def _mm_head(fu, w):
  def body(fu_ref, w_ref, o_ref):
    o_ref[...] = jnp.dot(fu_ref[...], w_ref[...],
                         preferred_element_type=jnp.float32)
  return pl.pallas_call(
      body,
      grid=(_GRID,),
      in_specs=[
          pl.BlockSpec((_RB, _C), lambda i: (i, 0)),
          pl.BlockSpec((_C, _C), lambda i: (0, 0)),
      ],
      out_specs=pl.BlockSpec((_RB, _C), lambda i: (i, 0)),
      out_shape=jax.ShapeDtypeStruct((_N, _C), jnp.float32),
  )(fu, w)


def _mm_mid(fu, fd, p, wsu, wn1, ws1, bu, b1):
  def body(fu_ref, fd_ref, p_ref, wsu_ref, wn1_ref, ws1_ref, bu_ref, b1_ref,
           z_ref, s_ref):
    x1 = (jnp.dot(fu_ref[...], wsu_ref[...],
                  preferred_element_type=jnp.float32)
          + p_ref[0] + p_ref[1] + bu_ref[...])
    fd = fd_ref[...]
    z_ref[...] = (
        jnp.dot(x1, wn1_ref[0:_C], preferred_element_type=jnp.float32)
        + jnp.dot(fd, wn1_ref[_C:2 * _C], preferred_element_type=jnp.float32))
    s_ref[...] = (
        jnp.dot(x1, ws1_ref[0:_C], preferred_element_type=jnp.float32)
        + jnp.dot(fd, ws1_ref[_C:2 * _C], preferred_element_type=jnp.float32)
        + b1_ref[...])
  return pl.pallas_call(
      body,
      grid=(_GRID,),
      in_specs=[
          pl.BlockSpec((_RB, _C), lambda i: (i, 0)),
          pl.BlockSpec((_RB, _C), lambda i: (i, 0)),
          pl.BlockSpec((_NC, _RB, _C), lambda i: (0, i, 0)),
          pl.BlockSpec((_C, _C), lambda i: (0, 0)),
          pl.BlockSpec((2 * _C, _C), lambda i: (0, 0)),
          pl.BlockSpec((2 * _C, _C), lambda i: (0, 0)),
          pl.BlockSpec((1, _C), lambda i: (0, 0)),
          pl.BlockSpec((1, _C), lambda i: (0, 0)),
      ],
      out_specs=[
          pl.BlockSpec((_RB, _C), lambda i: (i, 0)),
          pl.BlockSpec((_RB, _C), lambda i: (i, 0)),
      ],
      out_shape=[
          jax.ShapeDtypeStruct((_N, _C), jnp.float32),
          jax.ShapeDtypeStruct((_N, _C), jnp.float32),
      ],
  )(fu, fd, p, wsu, wn1, ws1, bu, b1)


def _mm_tail(s, q):
  def body(s_ref, q_ref, o_ref):
    o_ref[...] = jnp.maximum(s_ref[...] + q_ref[0] + q_ref[1], 0.0)
  return pl.pallas_call(
      body,
      grid=(_GRID,),
      in_specs=[
          pl.BlockSpec((_RB, _C), lambda i: (i, 0)),
          pl.BlockSpec((_NC, _RB, _C), lambda i: (0, i, 0)),
      ],
      out_specs=pl.BlockSpec((_RB, _C), lambda i: (i, 0)),
      out_shape=jax.ShapeDtypeStruct((_N, _C), jnp.float32),
  )(s, q)


def kernel(from_up, from_down, edge_index, W_self_up, W_nbr_up, b_up,
           W_self_1, W_nbr_1, b_1):
  npad_e = _EPAD - _E
  pad_src = jnp.arange(npad_e, dtype=jnp.int32) % _N
  pad_dst = _N + jnp.arange(npad_e, dtype=jnp.int32) % (_NPAD - _N)
  src = jnp.concatenate([edge_index[0], pad_src]).reshape(
      _NC * _NS, _NSEG, _SEGC, _B)
  dst = jnp.concatenate([edge_index[1], pad_dst]).reshape(
      _NC * _NS, _NSEG, _SEGC, _B)
  bu = b_up.reshape(1, _C)
  b1 = b_1.reshape(1, _C)

  u = _mm_head(from_up, W_nbr_up)
  p = _segsum_sc(u, src, dst)
  z, s = _mm_mid(from_up, from_down, p, W_self_up, W_nbr_1, W_self_1, bu, b1)
  q = _segsum_sc(z, src, dst)
  return _mm_tail(s, q)


# speedup vs baseline: 1.0413x; 1.0413x over previous
import functools

import jax
import jax.numpy as jnp
from jax import lax
from jax.experimental import pallas as pl
from jax.experimental.pallas import tpu as pltpu
from jax.experimental.pallas import tpu_sc as plsc

_N = 10000
_E = 320000
_C = 128
_NC = 2
_NS = 16
_NPAD = 10240
_ROWS_PER_TILE = _NPAD // _NS
_EDGES_PER_CORE = _E // _NC
_EDGES_PER_TILE = _EDGES_PER_CORE // _NS
_B = 112
_SEGC = 6
_NSEG = 15
_EPT = _NSEG * _SEGC * _B
_EPAD = _NC * _NS * _EPT


def _segsum_sc(x, src5, dst5):
  mesh = plsc.VectorSubcoreMesh(
      core_axis_name="c", subcore_axis_name="s", num_cores=_NC,
      num_subcores=_NS)

  @functools.partial(
      pl.kernel,
      out_type=jax.ShapeDtypeStruct((_NC, _NPAD, _C), jnp.float32),
      mesh=mesh,
      scratch_types=[
          pltpu.VMEM_SHARED((_NPAD, _C), jnp.float32),
          pltpu.VMEM((_SEGC, _B), jnp.int32),
          pltpu.VMEM((_SEGC, _B), jnp.int32),
          pltpu.VMEM((_SEGC, _B), jnp.int32),
          pltpu.VMEM((_SEGC, _B), jnp.int32),
          pltpu.VMEM((_B, _C), jnp.float32),
          pltpu.VMEM((_B, _C), jnp.float32),
          pltpu.VMEM((_B, _C), jnp.float32),
          [pltpu.SemaphoreType.DMA] * 10,
      ],
  )
  def k(x_hbm, src_hbm, dst_hbm, out_hbm, acc,
        sbuf0, sbuf1, dbuf0, dbuf1, rows0, rows1, rows2, sems):
    c = lax.axis_index("c")
    s = lax.axis_index("s")
    w = c * _NS + s
    row0 = s * _ROWS_PER_TILE
    sbuf = (sbuf0, sbuf1)
    dbuf = (dbuf0, dbuf1)
    rows = (rows0, rows1, rows2)
    issem = (sems[0], sems[1])
    idsem = (sems[2], sems[3])
    gsem = (sems[4], sems[5], sems[6])
    ssem = (sems[7], sems[8], sems[9])

    def seg_start(sb, g):
      pltpu.async_copy(src_hbm.at[w, g], sbuf[sb], issem[sb])
      pltpu.async_copy(dst_hbm.at[w, g], dbuf[sb], idsem[sb])

    def seg_wait(sb):
      pltpu.make_async_copy(src_hbm.at[w, 0], sbuf[sb], issem[sb]).wait()
      pltpu.make_async_copy(dst_hbm.at[w, 0], dbuf[sb], idsem[sb]).wait()

    def g_start(b, sb, k_):
      pltpu.async_copy(x_hbm.at[sbuf[sb].at[k_]], rows[b], gsem[b])

    def g_wait(b):
      pltpu.make_async_copy(x_hbm.at[sbuf[0].at[0]], rows[b], gsem[b]).wait()

    def s_start(b, sb, k_):
      pltpu.async_copy(rows[b], acc.at[dbuf[sb].at[k_]], ssem[b], add=True)

    def s_wait(b):
      pltpu.make_async_copy(rows[b], acc.at[dbuf[0].at[0]], ssem[b]).wait()

    seg_start(0, 0)

    zv = jnp.zeros((16,), jnp.float32)

    def zbody(i, carry):
      rows0[i // 8, pl.ds((i % 8) * 16, 16)] = zv
      return carry

    lax.fori_loop(0, _B * _C // 16, zbody, 0)
    for r in range(_ROWS_PER_TILE // _B):
      pltpu.sync_copy(rows0, acc.at[pl.ds(row0 + r * _B, _B)])
    _TAIL = _ROWS_PER_TILE % _B
    if _TAIL:
      pltpu.sync_copy(
          rows0.at[pl.ds(0, _TAIL)],
          acc.at[pl.ds(row0 + (_ROWS_PER_TILE // _B) * _B, _TAIL)])
    plsc.subcore_barrier()
    seg_wait(0)
    g_start(0, 0, 0)
    g_start(1, 0, 1)

    def chunks(sb, seg, first, last):
      for k_ in range(_SEGC):
        b = k_ % 3
        b1 = (k_ + 2) % 3
        g_wait(b)
        s_start(b, sb, k_)
        if k_ == 0 and first:
          pass
        else:
          s_wait(b1)
        if k_ == 0 and not last:
          seg_start(1 - sb, seg + 1)
        if k_ < _SEGC - 2:
          g_start(b1, sb, k_ + 2)
        elif not last:
          if k_ == _SEGC - 2:
            seg_wait(1 - sb)
            g_start(b1, 1 - sb, 0)
          else:
            g_start(b1, 1 - sb, 1)

    def body(gg, carry):
      for sb in (0, 1):
        chunks(sb, 2 * gg + sb, first=False, last=False)
      return carry

    chunks(0, 0, first=True, last=False)
    chunks(1, 1, first=False, last=False)
    lax.fori_loop(1, (_NSEG - 1) // 2, body, 0)
    chunks(0, _NSEG - 1, first=False, last=True)
    s_wait((_NSEG * _SEGC - 1) % 3)

    plsc.subcore_barrier()
    pltpu.sync_copy(acc.at[pl.ds(row0, _ROWS_PER_TILE)],
                    out_hbm.at[c, pl.ds(row0, _ROWS_PER_TILE)])

  return k(x, src5, dst5)


_RB = 5000
_GRID = _N // _RB


def _mm_head(fu, w):
  def body(fu_ref, w_ref, o_ref):
    o_ref[...] = jnp.dot(fu_ref[...], w_ref[...],
                         preferred_element_type=jnp.float32)
  return pl.pallas_call(
      body,
      grid=(_GRID,),
      in_specs=[
          pl.BlockSpec((_RB, _C), lambda i: (i, 0)),
          pl.BlockSpec((_C, _C), lambda i: (0, 0)),
      ],
      out_specs=pl.BlockSpec((_RB, _C), lambda i: (i, 0)),
      out_shape=jax.ShapeDtypeStruct((_N, _C), jnp.float32),
  )(fu, w)


def _mm_mid(fu, fd, p, wsu, wn1, ws1, bu, b1):
  def body(fu_ref, fd_ref, p_ref, wsu_ref, wn1_ref, ws1_ref, bu_ref, b1_ref,
           z_ref, s_ref):
    x1 = (jnp.dot(fu_ref[...], wsu_ref[...],
                  preferred_element_type=jnp.float32)
          + p_ref[0] + p_ref[1] + bu_ref[...])
    fd = fd_ref[...]
    z_ref[...] = (
        jnp.dot(x1, wn1_ref[0:_C], preferred_element_type=jnp.float32)
        + jnp.dot(fd, wn1_ref[_C:2 * _C], preferred_element_type=jnp.float32))
    s_ref[...] = (
        jnp.dot(x1, ws1_ref[0:_C], preferred_element_type=jnp.float32)
        + jnp.dot(fd, ws1_ref[_C:2 * _C], preferred_element_type=jnp.float32)
        + b1_ref[...])
  return pl.pallas_call(
      body,
      grid=(_GRID,),
      in_specs=[
          pl.BlockSpec((_RB, _C), lambda i: (i, 0)),
          pl.BlockSpec((_RB, _C), lambda i: (i, 0)),
          pl.BlockSpec((_NC, _RB, _C), lambda i: (0, i, 0)),
          pl.BlockSpec((_C, _C), lambda i: (0, 0)),
          pl.BlockSpec((2 * _C, _C), lambda i: (0, 0)),
          pl.BlockSpec((2 * _C, _C), lambda i: (0, 0)),
          pl.BlockSpec((1, _C), lambda i: (0, 0)),
          pl.BlockSpec((1, _C), lambda i: (0, 0)),
      ],
      out_specs=[
          pl.BlockSpec((_RB, _C), lambda i: (i, 0)),
          pl.BlockSpec((_RB, _C), lambda i: (i, 0)),
      ],
      out_shape=[
          jax.ShapeDtypeStruct((_N, _C), jnp.float32),
          jax.ShapeDtypeStruct((_N, _C), jnp.float32),
      ],
  )(fu, fd, p, wsu, wn1, ws1, bu, b1)


def _mm_tail(s, q):
  def body(s_ref, q_ref, o_ref):
    o_ref[...] = jnp.maximum(s_ref[...] + q_ref[0] + q_ref[1], 0.0)
  return pl.pallas_call(
      body,
      grid=(_GRID,),
      in_specs=[
          pl.BlockSpec((_RB, _C), lambda i: (i, 0)),
          pl.BlockSpec((_NC, _RB, _C), lambda i: (0, i, 0)),
      ],
      out_specs=pl.BlockSpec((_RB, _C), lambda i: (i, 0)),
      out_shape=jax.ShapeDtypeStruct((_N, _C), jnp.float32),
  )(s, q)


def kernel(from_up, from_down, edge_index, W_self_up, W_nbr_up, b_up,
           W_self_1, W_nbr_1, b_1):
  npad_e = _EPAD - _E
  pad_src = jnp.arange(npad_e, dtype=jnp.int32) % _N
  pad_dst = _N + jnp.arange(npad_e, dtype=jnp.int32) % (_NPAD - _N)
  src = jnp.concatenate([edge_index[0], pad_src]).reshape(
      _NC * _NS, _NSEG, _SEGC, _B)
  dst = jnp.concatenate([edge_index[1], pad_dst]).reshape(
      _NC * _NS, _NSEG, _SEGC, _B)
  bu = b_up.reshape(1, _C)
  b1 = b_1.reshape(1, _C)

  u = _mm_head(from_up, W_nbr_up)
  p = _segsum_sc(u, src, dst)
  z, s = _mm_mid(from_up, from_down, p, W_self_up, W_nbr_1, W_self_1, bu, b1)
  q = _segsum_sc(z, src, dst)
  return _mm_tail(s, q)
